# baseline (speedup 1.0000x reference)
import jax
import jax.numpy as jnp
from jax.experimental import pallas as pl
from jax.experimental.pallas import tpu as pltpu

K = 1024
D = 64
NC = 8


def _vq_body(c_ref, z_ref, cb_ref, zq_ref, qb_ref):
    H, W = z_ref.shape[2], z_ref.shape[3]
    HW = H * W
    z = z_ref[0].reshape(D, HW)
    cb = cb_ref[0]
    a = jnp.sum(z * z, axis=0, keepdims=True)
    e = jax.lax.dot_general(cb, z, (((1,), (0,)), ((), ())),
                            precision=jax.lax.Precision.DEFAULT)
    b2 = jnp.sum(cb * cb, axis=-1, keepdims=True)
    dists = a - 2.0 * e + b2
    m = jnp.min(dists, axis=0, keepdims=True)
    iota = jax.lax.broadcasted_iota(jnp.int32, dists.shape, 0)
    masked = jnp.where(dists == m, iota, K)
    idx = jnp.min(masked, axis=0, keepdims=True)
    onehot = (masked == idx).astype(jnp.bfloat16)
    cb_hi = cb.astype(jnp.bfloat16)
    cb_lo = (cb - cb_hi.astype(jnp.float32)).astype(jnp.bfloat16)
    chl = jnp.concatenate([cb_hi, cb_lo], axis=1)
    dn = (((0,), (0,)), ((), ()))
    qhl = jax.lax.dot_general(chl, onehot, dn,
                              preferred_element_type=jnp.float32)
    quant = qhl[:D] + qhl[D:]
    zq = z + (quant - z)
    zq_ref[0] = zq.reshape(D, H, W)
    qb_ref[0] = quant.reshape(D, H, W)


def kernel(z_e_x, C, emb_weight):
    B, Dd, H, W = z_e_x.shape
    grid_spec = pltpu.PrefetchScalarGridSpec(
        num_scalar_prefetch=1,
        grid=(B,),
        in_specs=[
            pl.BlockSpec((1, Dd, H, W), lambda b, c: (b, 0, 0, 0)),
            pl.BlockSpec((1, K, Dd), lambda b, c: (c[b], 0, 0)),
        ],
        out_specs=[
            pl.BlockSpec((1, Dd, H, W), lambda b, c: (b, 0, 0, 0)),
            pl.BlockSpec((1, Dd, H, W), lambda b, c: (b, 0, 0, 0)),
        ],
    )
    zq, qb = pl.pallas_call(
        _vq_body,
        grid_spec=grid_spec,
        out_shape=[
            jax.ShapeDtypeStruct((B, Dd, H, W), jnp.float32),
            jax.ShapeDtypeStruct((B, Dd, H, W), jnp.float32),
        ],
    )(C, z_e_x, emb_weight)
    return zq, qb


# speedup vs baseline: 2.0595x; 2.0595x over previous
import jax
import jax.numpy as jnp
from jax.experimental import pallas as pl
from jax.experimental.pallas import tpu as pltpu

K = 1024
D = 64
NC = 8


def _vq_body(c_ref, z_ref, ct_ref, zq_ref, qb_ref):
    z = z_ref[0]
    ct = ct_ref[0]
    a = jnp.sum(z * z, axis=1, keepdims=True)
    e = jax.lax.dot_general(z, ct, (((1,), (0,)), ((), ())),
                            precision=jax.lax.Precision.DEFAULT)
    b2 = jnp.sum(ct * ct, axis=0, keepdims=True)
    dists = a - 2.0 * e + b2
    m = jnp.min(dists, axis=1, keepdims=True)
    iota = jax.lax.broadcasted_iota(jnp.int32, dists.shape, 1)
    masked = jnp.where(dists == m, iota, K)
    idx = jnp.min(masked, axis=1, keepdims=True)
    onehot = (masked == idx).astype(jnp.bfloat16)
    ct_hi = ct.astype(jnp.bfloat16)
    ct_lo = (ct - ct_hi.astype(jnp.float32)).astype(jnp.bfloat16)
    chl = jnp.concatenate([ct_hi, ct_lo], axis=0)
    dn = (((1,), (1,)), ((), ()))
    qhl = jax.lax.dot_general(onehot, chl, dn,
                              preferred_element_type=jnp.float32)
    quant = qhl[:, :D] + qhl[:, D:]
    zq_ref[0] = z + (quant - z)
    qb_ref[0] = quant


def kernel(z_e_x, C, emb_weight):
    B, Dd, H, W = z_e_x.shape
    HW = H * W
    z = jnp.transpose(z_e_x, (0, 2, 3, 1)).reshape(B, HW, Dd)
    ct_all = jnp.transpose(emb_weight, (0, 2, 1))
    grid_spec = pltpu.PrefetchScalarGridSpec(
        num_scalar_prefetch=1,
        grid=(B,),
        in_specs=[
            pl.BlockSpec((1, HW, Dd), lambda b, c: (b, 0, 0)),
            pl.BlockSpec((1, Dd, K), lambda b, c: (c[b], 0, 0)),
        ],
        out_specs=[
            pl.BlockSpec((1, HW, Dd), lambda b, c: (b, 0, 0)),
            pl.BlockSpec((1, HW, Dd), lambda b, c: (b, 0, 0)),
        ],
    )
    zq, qb = pl.pallas_call(
        _vq_body,
        grid_spec=grid_spec,
        out_shape=[
            jax.ShapeDtypeStruct((B, HW, Dd), jnp.float32),
            jax.ShapeDtypeStruct((B, HW, Dd), jnp.float32),
        ],
    )(C, z, ct_all)
    z_q_x = jnp.transpose(zq.reshape(B, H, W, Dd), (0, 3, 1, 2))
    z_q_x_bar = jnp.transpose(qb.reshape(B, H, W, Dd), (0, 3, 1, 2))
    return z_q_x, z_q_x_bar


# speedup vs baseline: 2.1499x; 1.0439x over previous
import jax
import jax.numpy as jnp
from jax.experimental import pallas as pl
from jax.experimental.pallas import tpu as pltpu

K = 1024
D = 64
NC = 8


def _vq_one(z, ct):
    a = jnp.sum(z * z, axis=1, keepdims=True)
    e = jax.lax.dot_general(z, ct, (((1,), (0,)), ((), ())),
                            precision=jax.lax.Precision.DEFAULT)
    b2 = jnp.sum(ct * ct, axis=0, keepdims=True)
    dists = a - 2.0 * e + b2
    m = jnp.min(dists, axis=1, keepdims=True)
    iota = jax.lax.broadcasted_iota(jnp.int32, dists.shape, 1)
    masked = jnp.where(dists == m, iota, K)
    idx = jnp.min(masked, axis=1, keepdims=True)
    onehot = (masked == idx).astype(jnp.bfloat16)
    ct_hi = ct.astype(jnp.bfloat16)
    ct_lo = (ct - ct_hi.astype(jnp.float32)).astype(jnp.bfloat16)
    chl = jnp.concatenate([ct_hi, ct_lo], axis=0)
    dn = (((1,), (1,)), ((), ()))
    qhl = jax.lax.dot_general(onehot, chl, dn,
                              preferred_element_type=jnp.float32)
    quant = qhl[:, :D] + qhl[:, D:]
    return z + (quant - z), quant


def _vq_body(c_ref, z_ref, ct0_ref, ct1_ref, zq_ref, qb_ref):
    zq0, qb0 = _vq_one(z_ref[0], ct0_ref[0])
    zq_ref[0] = zq0
    qb_ref[0] = qb0
    zq1, qb1 = _vq_one(z_ref[1], ct1_ref[0])
    zq_ref[1] = zq1
    qb_ref[1] = qb1


def kernel(z_e_x, C, emb_weight):
    B, Dd, H, W = z_e_x.shape
    HW = H * W
    z = jnp.transpose(z_e_x, (0, 2, 3, 1)).reshape(B, HW, Dd)
    ct_all = jnp.transpose(emb_weight, (0, 2, 1))
    grid_spec = pltpu.PrefetchScalarGridSpec(
        num_scalar_prefetch=1,
        grid=(B // 2,),
        in_specs=[
            pl.BlockSpec((2, HW, Dd), lambda b, c: (b, 0, 0)),
            pl.BlockSpec((1, Dd, K), lambda b, c: (c[2 * b], 0, 0)),
            pl.BlockSpec((1, Dd, K), lambda b, c: (c[2 * b + 1], 0, 0)),
        ],
        out_specs=[
            pl.BlockSpec((2, HW, Dd), lambda b, c: (b, 0, 0)),
            pl.BlockSpec((2, HW, Dd), lambda b, c: (b, 0, 0)),
        ],
    )
    zq, qb = pl.pallas_call(
        _vq_body,
        grid_spec=grid_spec,
        out_shape=[
            jax.ShapeDtypeStruct((B, HW, Dd), jnp.float32),
            jax.ShapeDtypeStruct((B, HW, Dd), jnp.float32),
        ],
    )(C, z, ct_all, ct_all)
    z_q_x = jnp.transpose(zq.reshape(B, H, W, Dd), (0, 3, 1, 2))
    z_q_x_bar = jnp.transpose(qb.reshape(B, H, W, Dd), (0, 3, 1, 2))
    return z_q_x, z_q_x_bar


# speedup vs baseline: 2.4488x; 1.1391x over previous
import jax
import jax.numpy as jnp
from jax.experimental import pallas as pl
from jax.experimental.pallas import tpu as pltpu

K = 1024
D = 64
NC = 8


def _vq_one(z, ct):
    a = jnp.sum(z * z, axis=1, keepdims=True)
    e = jax.lax.dot_general(z, ct, (((1,), (0,)), ((), ())),
                            precision=jax.lax.Precision.DEFAULT)
    b2 = jnp.sum(ct * ct, axis=0, keepdims=True)
    dists = a - 2.0 * e + b2
    m = jnp.min(dists, axis=1, keepdims=True)
    iota = jax.lax.broadcasted_iota(jnp.int32, dists.shape, 1)
    masked = jnp.where(dists == m, iota, K)
    idx = jnp.min(masked, axis=1, keepdims=True)
    onehot = (masked == idx).astype(jnp.bfloat16)
    ct_hi = ct.astype(jnp.bfloat16)
    ct_lo = (ct - ct_hi.astype(jnp.float32)).astype(jnp.bfloat16)
    chl = jnp.concatenate([ct_hi, ct_lo], axis=0)
    dn = (((1,), (1,)), ((), ()))
    qhl = jax.lax.dot_general(onehot, chl, dn,
                              preferred_element_type=jnp.float32)
    quant = qhl[:, :D] + qhl[:, D:]
    return z + (quant - z), quant


def _vq_body(c_ref, z_ref, ct0_ref, ct1_ref, ct2_ref, ct3_ref, zq_ref, qb_ref):
    for i, ct_ref in enumerate((ct0_ref, ct1_ref, ct2_ref, ct3_ref)):
        zqi, qbi = _vq_one(z_ref[i], ct_ref[0])
        zq_ref[i] = zqi
        qb_ref[i] = qbi


def kernel(z_e_x, C, emb_weight):
    B, Dd, H, W = z_e_x.shape
    HW = H * W
    z = jnp.transpose(z_e_x, (0, 2, 3, 1)).reshape(B, HW, Dd)
    ct_all = jnp.transpose(emb_weight, (0, 2, 1))
    grid_spec = pltpu.PrefetchScalarGridSpec(
        num_scalar_prefetch=1,
        grid=(B // 4,),
        in_specs=[
            pl.BlockSpec((4, HW, Dd), lambda b, c: (b, 0, 0)),
            pl.BlockSpec((1, Dd, K), lambda b, c: (c[4 * b], 0, 0)),
            pl.BlockSpec((1, Dd, K), lambda b, c: (c[4 * b + 1], 0, 0)),
            pl.BlockSpec((1, Dd, K), lambda b, c: (c[4 * b + 2], 0, 0)),
            pl.BlockSpec((1, Dd, K), lambda b, c: (c[4 * b + 3], 0, 0)),
        ],
        out_specs=[
            pl.BlockSpec((4, HW, Dd), lambda b, c: (b, 0, 0)),
            pl.BlockSpec((4, HW, Dd), lambda b, c: (b, 0, 0)),
        ],
    )
    zq, qb = pl.pallas_call(
        _vq_body,
        grid_spec=grid_spec,
        out_shape=[
            jax.ShapeDtypeStruct((B, HW, Dd), jnp.float32),
            jax.ShapeDtypeStruct((B, HW, Dd), jnp.float32),
        ],
    )(C, z, ct_all, ct_all, ct_all, ct_all)
    z_q_x = jnp.transpose(zq.reshape(B, H, W, Dd), (0, 3, 1, 2))
    z_q_x_bar = jnp.transpose(qb.reshape(B, H, W, Dd), (0, 3, 1, 2))
    return z_q_x, z_q_x_bar


# speedup vs baseline: 2.4507x; 1.0008x over previous
import jax
import jax.numpy as jnp
from jax.experimental import pallas as pl
from jax.experimental.pallas import tpu as pltpu

K = 1024
D = 64
NC = 8


def _vq_one(z, ct):
    a = jnp.sum(z * z, axis=1, keepdims=True)
    e = jax.lax.dot_general(z, ct, (((1,), (0,)), ((), ())),
                            precision=jax.lax.Precision.DEFAULT)
    b2 = jnp.sum(ct * ct, axis=0, keepdims=True)
    dists = a - 2.0 * e + b2
    m = jnp.min(dists, axis=1, keepdims=True)
    iota = jax.lax.broadcasted_iota(jnp.int32, dists.shape, 1)
    masked = jnp.where(dists == m, iota, K)
    idx = jnp.min(masked, axis=1, keepdims=True)
    onehot = (masked == idx).astype(jnp.bfloat16)
    ct_hi = ct.astype(jnp.bfloat16)
    ct_lo = (ct - ct_hi.astype(jnp.float32)).astype(jnp.bfloat16)
    chl = jnp.concatenate([ct_hi, ct_lo], axis=0)
    dn = (((1,), (1,)), ((), ()))
    qhl = jax.lax.dot_general(onehot, chl, dn,
                              preferred_element_type=jnp.float32)
    quant = qhl[:, :D] + qhl[:, D:]
    return z + (quant - z), quant


def _vq_body(c_ref, z_ref, ct0_ref, ct1_ref, ct2_ref, ct3_ref,
             ct4_ref, ct5_ref, ct6_ref, ct7_ref, zq_ref, qb_ref):
    for i, ct_ref in enumerate((ct0_ref, ct1_ref, ct2_ref, ct3_ref,
                                ct4_ref, ct5_ref, ct6_ref, ct7_ref)):
        zqi, qbi = _vq_one(z_ref[i], ct_ref[0])
        zq_ref[i] = zqi
        qb_ref[i] = qbi


def kernel(z_e_x, C, emb_weight):
    B, Dd, H, W = z_e_x.shape
    HW = H * W
    z = jnp.transpose(z_e_x, (0, 2, 3, 1)).reshape(B, HW, Dd)
    ct_all = jnp.transpose(emb_weight, (0, 2, 1))
    grid_spec = pltpu.PrefetchScalarGridSpec(
        num_scalar_prefetch=1,
        grid=(B // 8,),
        in_specs=[pl.BlockSpec((8, HW, Dd), lambda b, c: (b, 0, 0))] + [
            pl.BlockSpec((1, Dd, K),
                         (lambda j: lambda b, c: (c[8 * b + j], 0, 0))(j))
            for j in range(8)
        ],
        out_specs=[
            pl.BlockSpec((8, HW, Dd), lambda b, c: (b, 0, 0)),
            pl.BlockSpec((8, HW, Dd), lambda b, c: (b, 0, 0)),
        ],
    )
    zq, qb = pl.pallas_call(
        _vq_body,
        grid_spec=grid_spec,
        out_shape=[
            jax.ShapeDtypeStruct((B, HW, Dd), jnp.float32),
            jax.ShapeDtypeStruct((B, HW, Dd), jnp.float32),
        ],
    )(C, z, *([ct_all] * 8))
    z_q_x = jnp.transpose(zq.reshape(B, H, W, Dd), (0, 3, 1, 2))
    z_q_x_bar = jnp.transpose(qb.reshape(B, H, W, Dd), (0, 3, 1, 2))
    return z_q_x, z_q_x_bar


# speedup vs baseline: 2.4768x; 1.0106x over previous
import jax
import jax.numpy as jnp
from jax.experimental import pallas as pl
from jax.experimental.pallas import tpu as pltpu

K = 1024
D = 64
NC = 8


def _vq_one(z, ct):
    a = jnp.sum(z * z, axis=1, keepdims=True)
    e = jax.lax.dot_general(z, ct, (((1,), (0,)), ((), ())),
                            precision=jax.lax.Precision.DEFAULT)
    b2 = jnp.sum(ct * ct, axis=0, keepdims=True)
    dists = a - 2.0 * e + b2
    m = jnp.min(dists, axis=1, keepdims=True)
    iota = jax.lax.broadcasted_iota(jnp.int32, dists.shape, 1)
    masked = jnp.where(dists == m, iota, K)
    idx = jnp.min(masked, axis=1, keepdims=True)
    onehot = (masked == idx).astype(jnp.bfloat16)
    ct_hi = ct.astype(jnp.bfloat16)
    dn = (((1,), (1,)), ((), ()))
    quant = jax.lax.dot_general(onehot, ct_hi, dn,
                                preferred_element_type=jnp.float32)
    return z + (quant - z), quant


def _vq_body(c_ref, z_ref, ct0_ref, ct1_ref, ct2_ref, ct3_ref,
             ct4_ref, ct5_ref, ct6_ref, ct7_ref, zq_ref, qb_ref):
    for i, ct_ref in enumerate((ct0_ref, ct1_ref, ct2_ref, ct3_ref,
                                ct4_ref, ct5_ref, ct6_ref, ct7_ref)):
        zqi, qbi = _vq_one(z_ref[i], ct_ref[0])
        zq_ref[i] = zqi
        qb_ref[i] = qbi


def kernel(z_e_x, C, emb_weight):
    B, Dd, H, W = z_e_x.shape
    HW = H * W
    z = jnp.transpose(z_e_x, (0, 2, 3, 1)).reshape(B, HW, Dd)
    ct_all = jnp.transpose(emb_weight, (0, 2, 1))
    grid_spec = pltpu.PrefetchScalarGridSpec(
        num_scalar_prefetch=1,
        grid=(B // 8,),
        in_specs=[pl.BlockSpec((8, HW, Dd), lambda b, c: (b, 0, 0))] + [
            pl.BlockSpec((1, Dd, K),
                         (lambda j: lambda b, c: (c[8 * b + j], 0, 0))(j))
            for j in range(8)
        ],
        out_specs=[
            pl.BlockSpec((8, HW, Dd), lambda b, c: (b, 0, 0)),
            pl.BlockSpec((8, HW, Dd), lambda b, c: (b, 0, 0)),
        ],
    )
    zq, qb = pl.pallas_call(
        _vq_body,
        grid_spec=grid_spec,
        out_shape=[
            jax.ShapeDtypeStruct((B, HW, Dd), jnp.float32),
            jax.ShapeDtypeStruct((B, HW, Dd), jnp.float32),
        ],
    )(C, z, *([ct_all] * 8))
    z_q_x = jnp.transpose(zq.reshape(B, H, W, Dd), (0, 3, 1, 2))
    z_q_x_bar = jnp.transpose(qb.reshape(B, H, W, Dd), (0, 3, 1, 2))
    return z_q_x, z_q_x_bar
